# baseline (speedup 1.0000x reference)
import functools

import jax
import jax.numpy as jnp
from jax import lax
from jax.experimental import pallas as pl
from jax.experimental.pallas import tpu as pltpu
from jax.experimental.pallas import tpu_sc as plsc

VOCAB = 100000
D_MODEL = 1024
B = 4
T = 4096

_info = plsc.get_sparse_core_info()
_NC, _NS = _info.num_cores, _info.num_subcores
_NW = _NC * _NS

_N = B * T
_BPW = _N // _NW
_CMAX = 32
_NBUF = 3
_SIZES = [8, 8, 16] + [32] * 14 + [16, 8, 8]
assert sum(_SIZES) == _BPW and all(s % 8 == 0 and s <= _CMAX for s in _SIZES)
_OFFS = [sum(_SIZES[:i]) for i in range(len(_SIZES))]
_NCHUNK = len(_SIZES)
_HEAD = _SIZES[0] + _SIZES[1] + _SIZES[2]


def _mesh_kernel():
    mesh = plsc.VectorSubcoreMesh(core_axis_name="c", subcore_axis_name="s")

    @functools.partial(
        pl.kernel,
        mesh=mesh,
        out_type=jax.ShapeDtypeStruct((_N, D_MODEL), jnp.float32),
        scratch_types=(
            [pltpu.VMEM((_BPW,), jnp.int32)]
            + [pltpu.VMEM((_CMAX, D_MODEL), jnp.float32)] * _NBUF
            + [pltpu.SemaphoreType.DMA] * (2 * _NBUF)
        ),
    )
    def gather_kernel(idx_hbm, table_hbm, out_hbm, idx_v, *bufs_and_sems):
        bufs = bufs_and_sems[:_NBUF]
        gsems = bufs_and_sems[_NBUF:2 * _NBUF]
        wsems = bufs_and_sems[2 * _NBUF:]
        wid = lax.axis_index("s") * _NC + lax.axis_index("c")
        base = wid * _BPW

        def start_gather(c, b):
            sz = _SIZES[c]
            return pltpu.async_copy(
                table_hbm.at[idx_v.at[pl.ds(_OFFS[c], sz)]],
                bufs[b].at[pl.ds(0, sz)],
                gsems[b])

        pltpu.sync_copy(idx_hbm.at[pl.ds(base, _HEAD)],
                        idx_v.at[pl.ds(0, _HEAD)])
        g = [None] * _NBUF
        w = [None] * _NBUF
        for c in range(_NBUF - 1):
            g[c] = start_gather(c, c)
        pltpu.sync_copy(idx_hbm.at[pl.ds(base + _HEAD, _BPW - _HEAD)],
                        idx_v.at[pl.ds(_HEAD, _BPW - _HEAD)])

        for c in range(_NCHUNK):
            b = c % _NBUF
            nc = c + _NBUF - 1
            if nc < _NCHUNK:
                nb = nc % _NBUF
                if w[nb] is not None:
                    w[nb].wait()
                g[nb] = start_gather(nc, nb)
            g[b].wait()
            w[b] = pltpu.async_copy(
                bufs[b].at[pl.ds(0, _SIZES[c])],
                out_hbm.at[pl.ds(base + _OFFS[c], _SIZES[c])],
                wsems[b])
        for c in range(max(0, _NCHUNK - _NBUF), _NCHUNK):
            w[c % _NBUF].wait()

    return gather_kernel


_GATHER = _mesh_kernel()


def kernel(x_ids, table):
    ids = x_ids.reshape(_N)
    out = _GATHER(ids, table)
    return out.reshape(B, T, D_MODEL)


# speedup vs baseline: 1.0053x; 1.0053x over previous
import functools

import jax
import jax.numpy as jnp
from jax import lax
from jax.experimental import pallas as pl
from jax.experimental.pallas import tpu as pltpu
from jax.experimental.pallas import tpu_sc as plsc

VOCAB = 100000
D_MODEL = 1024
B = 4
T = 4096

_info = plsc.get_sparse_core_info()
_NC, _NS = _info.num_cores, _info.num_subcores
_NW = _NC * _NS

_N = B * T
_BPW = _N // _NW
_C = 16
_NCHUNK = _BPW // _C
_NBUF = 7


def _mesh_kernel():
    mesh = plsc.VectorSubcoreMesh(core_axis_name="c", subcore_axis_name="s")

    @functools.partial(
        pl.kernel,
        mesh=mesh,
        out_type=jax.ShapeDtypeStruct((_N, D_MODEL), jnp.float32),
        scratch_types=(
            [pltpu.VMEM((_NCHUNK, _C), jnp.int32)]
            + [pltpu.VMEM((_C, D_MODEL), jnp.float32)] * _NBUF
            + [pltpu.SemaphoreType.DMA] * (2 * _NBUF)
        ),
    )
    def gather_kernel(idx_hbm, table_hbm, out_hbm, idx_v, *bufs_and_sems):
        bufs = bufs_and_sems[:_NBUF]
        gsems = bufs_and_sems[_NBUF:2 * _NBUF]
        wsems = bufs_and_sems[2 * _NBUF:]
        wid = lax.axis_index("s") * _NC + lax.axis_index("c")
        base = wid * _BPW
        pltpu.sync_copy(idx_hbm.at[wid], idx_v)

        g = [None] * _NBUF
        w = [None] * _NBUF
        for c in range(min(_NBUF - 1, _NCHUNK)):
            b = c % _NBUF
            g[b] = pltpu.async_copy(table_hbm.at[idx_v.at[c]], bufs[b], gsems[b])
        for c in range(_NCHUNK):
            b = c % _NBUF
            nc = c + _NBUF - 1
            if nc < _NCHUNK:
                nb = nc % _NBUF
                if w[nb] is not None:
                    w[nb].wait()
                g[nb] = pltpu.async_copy(
                    table_hbm.at[idx_v.at[nc]], bufs[nb], gsems[nb])
            g[b].wait()
            w[b] = pltpu.async_copy(
                bufs[b], out_hbm.at[pl.ds(base + c * _C, _C)], wsems[b])
        for c in range(max(0, _NCHUNK - _NBUF), _NCHUNK):
            w[c % _NBUF].wait()

    return gather_kernel


_GATHER = _mesh_kernel()


def kernel(x_ids, table):
    ids = x_ids.reshape(_NW, _NCHUNK, _C)
    out = _GATHER(ids, table)
    return out.reshape(B, T, D_MODEL)


# speedup vs baseline: 1.0074x; 1.0021x over previous
import functools

import jax
import jax.numpy as jnp
from jax import lax
from jax.experimental import pallas as pl
from jax.experimental.pallas import tpu as pltpu
from jax.experimental.pallas import tpu_sc as plsc

VOCAB = 100000
D_MODEL = 1024
B = 4
T = 4096

try:
    _info = plsc.get_sparse_core_info()
    _NC, _NS = _info.num_cores, _info.num_subcores
except Exception:
    _NC, _NS = 2, 16
_NW = _NC * _NS

_N = B * T
_BPW = _N // _NW
_C = 16
_NCHUNK = _BPW // _C
_NBUF = 7


def _mesh_kernel():
    mesh = plsc.VectorSubcoreMesh(core_axis_name="c", subcore_axis_name="s")

    @functools.partial(
        pl.kernel,
        mesh=mesh,
        out_type=jax.ShapeDtypeStruct((_N, D_MODEL), jnp.float32),
        scratch_types=(
            [pltpu.VMEM((_NCHUNK, _C), jnp.int32)]
            + [pltpu.VMEM((_C, D_MODEL), jnp.float32)] * _NBUF
            + [pltpu.SemaphoreType.DMA] * (2 * _NBUF)
        ),
    )
    def gather_kernel(idx_hbm, table_hbm, out_hbm, idx_v, *bufs_and_sems):
        bufs = bufs_and_sems[:_NBUF]
        gsems = bufs_and_sems[_NBUF:2 * _NBUF]
        wsems = bufs_and_sems[2 * _NBUF:]
        wid = lax.axis_index("s") * _NC + lax.axis_index("c")
        base = wid * _BPW
        pltpu.sync_copy(idx_hbm.at[wid], idx_v)

        g = [None] * _NBUF
        w = [None] * _NBUF
        for c in range(min(_NBUF - 1, _NCHUNK)):
            b = c % _NBUF
            g[b] = pltpu.async_copy(table_hbm.at[idx_v.at[c]], bufs[b], gsems[b])
        for c in range(_NCHUNK):
            b = c % _NBUF
            nc = c + _NBUF - 1
            if nc < _NCHUNK:
                nb = nc % _NBUF
                if w[nb] is not None:
                    w[nb].wait()
                g[nb] = pltpu.async_copy(
                    table_hbm.at[idx_v.at[nc]], bufs[nb], gsems[nb])
            g[b].wait()
            w[b] = pltpu.async_copy(
                bufs[b], out_hbm.at[pl.ds(base + c * _C, _C)], wsems[b])
        for c in range(max(0, _NCHUNK - _NBUF), _NCHUNK):
            w[c % _NBUF].wait()

    return gather_kernel


_CACHE = {}


def kernel(x_ids, table):
    if "gather" not in _CACHE:
        _CACHE["gather"] = _mesh_kernel()
    ids = x_ids.reshape(_NW, _NCHUNK, _C)
    out = _CACHE["gather"](ids, table)
    return out.reshape(B, T, D_MODEL)
